# baseline (speedup 1.0000x reference)
import functools

import jax
import jax.numpy as jnp
from jax import lax
from jax.experimental import pallas as pl
from jax.experimental.pallas import tpu as pltpu
from jax.experimental.pallas import tpu_sc as plsc

SC_ROWS = 2560
ST_CHUNK = 16
TC_BLK = 512


def _sc_stage(table, seq_len, emb):
    info = plsc.get_sparse_core_info()
    nw = info.num_cores * info.num_subcores
    start = seq_len - SC_ROWS
    rows_per = SC_ROWS // nw
    n = rows_per // ST_CHUNK
    mesh = plsc.VectorSubcoreMesh(core_axis_name="c", subcore_axis_name="s")

    @functools.partial(
        pl.kernel,
        mesh=mesh,
        out_type=jax.ShapeDtypeStruct((seq_len, emb), table.dtype),
        scratch_types=[pltpu.VMEM((2, ST_CHUNK, emb), table.dtype)]
        + [pltpu.SemaphoreType.DMA] * 4,
    )
    def sc_copy(table_hbm, out_hbm, buf, *sems):
        wid = lax.axis_index("s") * info.num_cores + lax.axis_index("c")
        base = start + wid * rows_per
        sin, sout = sems[0:2], sems[2:4]

        def in_copy(g, b):
            return pltpu.make_async_copy(
                table_hbm.at[pl.ds(base + g * ST_CHUNK, ST_CHUNK)],
                buf.at[b],
                sin[b],
            )

        def out_copy(g, b):
            return pltpu.make_async_copy(
                buf.at[b],
                out_hbm.at[pl.ds(base + g * ST_CHUNK, ST_CHUNK)],
                sout[b],
            )

        in_copy(0, 0).start()
        for g in range(n):
            b = g & 1
            in_copy(g, b).wait()
            if g + 1 < n:
                if g >= 1:
                    out_copy(g - 1, 1 - b).wait()
                in_copy(g + 1, 1 - b).start()
            out_copy(g, b).start()
        out_copy(n - 1, (n - 1) & 1).wait()
        if n >= 2:
            out_copy(n - 2, (n - 2) & 1).wait()

    return sc_copy(table)


def _tc_stage(partial_out, table, seq_len, emb):
    tc_rows = seq_len - SC_ROWS

    def body(acc, src, dst):
        del acc
        dst[...] = src[...]

    return pl.pallas_call(
        body,
        grid=(tc_rows // TC_BLK,),
        in_specs=[
            pl.BlockSpec(memory_space=pltpu.MemorySpace.HBM),
            pl.BlockSpec((TC_BLK, emb), lambda i: (i, 0)),
        ],
        out_specs=pl.BlockSpec((TC_BLK, emb), lambda i: (i, 0)),
        out_shape=jax.ShapeDtypeStruct((seq_len, emb), table.dtype),
        input_output_aliases={0: 0},
    )(partial_out, table)


def kernel(x, table):
    bs, seq_len = x.shape
    num_rows, emb = table.shape
    partial_out = _sc_stage(table, seq_len, emb)
    return _tc_stage(partial_out, table, seq_len, emb)


# speedup vs baseline: 1.0634x; 1.0634x over previous
import functools

import jax
import jax.numpy as jnp
from jax import lax
from jax.experimental import pallas as pl
from jax.experimental.pallas import tpu as pltpu
from jax.experimental.pallas import tpu_sc as plsc

ST_CHUNKS = (56, 56, 56, 56, 32)


def kernel(x, table):
    bs, seq_len = x.shape
    num_rows, emb = table.shape

    info = plsc.get_sparse_core_info()
    nw = info.num_cores * info.num_subcores
    rows_per = seq_len // nw
    assert rows_per == sum(ST_CHUNKS)
    offs = [sum(ST_CHUNKS[:i]) for i in range(len(ST_CHUNKS))]
    n = len(ST_CHUNKS)
    bufrows = max(ST_CHUNKS)
    mesh = plsc.VectorSubcoreMesh(core_axis_name="c", subcore_axis_name="s")

    @functools.partial(
        pl.kernel,
        mesh=mesh,
        out_type=jax.ShapeDtypeStruct((seq_len, emb), table.dtype),
        scratch_types=[pltpu.VMEM((2, bufrows, emb), table.dtype)]
        + [pltpu.SemaphoreType.DMA] * 4,
    )
    def positional_lookup(table_hbm, out_hbm, buf, *sems):
        wid = lax.axis_index("s") * info.num_cores + lax.axis_index("c")
        base = wid * rows_per
        sin, sout = sems[0:2], sems[2:4]

        def in_copy(g, b):
            return pltpu.make_async_copy(
                table_hbm.at[pl.ds(base + offs[g], ST_CHUNKS[g])],
                buf.at[b, pl.ds(0, ST_CHUNKS[g])],
                sin[b],
            )

        def out_copy(g, b):
            return pltpu.make_async_copy(
                buf.at[b, pl.ds(0, ST_CHUNKS[g])],
                out_hbm.at[pl.ds(base + offs[g], ST_CHUNKS[g])],
                sout[b],
            )

        in_copy(0, 0).start()
        for g in range(n):
            b = g & 1
            in_copy(g, b).wait()
            if g + 1 < n:
                if g >= 1:
                    out_copy(g - 1, 1 - b).wait()
                in_copy(g + 1, 1 - b).start()
            out_copy(g, b).start()
        out_copy(n - 1, (n - 1) & 1).wait()
        if n >= 2:
            out_copy(n - 2, (n - 2) & 1).wait()

    return positional_lookup(table)
